# Initial kernel scaffold; baseline (speedup 1.0000x reference)
import jax
import jax.numpy as jnp
from jax.experimental import pallas as pl


def kernel(h_node, x, ptr, W, b):
    raise NotImplementedError("write your pallas kernel here")


# speedup vs baseline: 100.4692x; 100.4692x over previous
import jax
import jax.numpy as jnp
from jax.experimental import pallas as pl
from jax.experimental.pallas import tpu as pltpu

_SMAX = 32


def _pick_block(n):
    for cand in (2560, 1280, 640, 320, 160, 80, 40, 16, 8):
        if n % cand == 0:
            return cand
    return n


def _body(ptr_ref, x_ref, h_ref, w_ref, b_ref, out_ref, *, K, F, D, B):
    k = pl.program_id(0)

    @pl.when(k == 0)
    def _init():
        out_ref[...] = jnp.zeros_like(out_ref)

    r0 = k * K
    z = jnp.dot(x_ref[...], w_ref[:F], preferred_element_type=jnp.float32)
    z = z + jnp.dot(h_ref[...], w_ref[F:], preferred_element_type=jnp.float32)
    z = z + b_ref[...]
    z = jnp.where(z >= 0, z, 0.2 * z)

    ptr_all = ptr_ref[...]

    def seg_of(r):
        return jnp.sum((ptr_all <= r).astype(jnp.int32)) - 1

    s0 = seg_of(r0)
    s1 = seg_of(r0 + K - 1)
    nchunks = (s1 - s0 + _SMAX) // _SMAX

    rows = r0 + jax.lax.broadcasted_iota(jnp.int32, (_SMAX, K), 1)

    def chunk(j, carry):
        base = s0 + j * _SMAX
        bounds = ptr_ref[pl.ds(base, _SMAX + 1), :]
        lo = bounds[:_SMAX]
        hi = bounds[1:]
        oh = ((rows >= lo) & (rows < hi)).astype(jnp.float32)
        part = jnp.dot(oh, z, preferred_element_type=jnp.float32)
        out_ref[pl.ds(base, _SMAX), :] += part
        return carry

    jax.lax.fori_loop(0, nchunks, chunk, 0)


def kernel(h_node, x, ptr, W, b):
    N, F = x.shape
    D = h_node.shape[1]
    B = ptr.shape[0] - 1
    K = _pick_block(N)
    G = N // K

    b_pad = -(B + _SMAX) % 8
    B_pad = B + _SMAX + b_pad
    P = -(B + 1 + _SMAX) % 8 + (B + 1 + _SMAX)
    ptr_pad = jnp.concatenate(
        [ptr.astype(jnp.int32), jnp.full((P - (B + 1),), N, jnp.int32)]
    ).reshape(P, 1)

    out = pl.pallas_call(
        lambda *refs: _body(*refs, K=K, F=F, D=D, B=B),
        grid=(G,),
        in_specs=[
            pl.BlockSpec((P, 1), lambda k: (0, 0)),
            pl.BlockSpec((K, F), lambda k: (k, 0)),
            pl.BlockSpec((K, D), lambda k: (k, 0)),
            pl.BlockSpec((F + D, D), lambda k: (0, 0)),
            pl.BlockSpec((1, D), lambda k: (0, 0)),
        ],
        out_specs=pl.BlockSpec((B_pad, D), lambda k: (0, 0)),
        out_shape=jax.ShapeDtypeStruct((B_pad, D), jnp.float32),
        compiler_params=pltpu.CompilerParams(
            dimension_semantics=("arbitrary",),
        ),
    )(ptr_pad, x, h_node, W, b.reshape(1, D))
    return out[:B]


# speedup vs baseline: 158.5899x; 1.5785x over previous
import jax
import jax.numpy as jnp
from jax.experimental import pallas as pl
from jax.experimental.pallas import tpu as pltpu

_SMAX = 32


def _pick_block(n):
    for cand in (2560, 1280, 640, 320, 160, 80, 40, 16, 8):
        if n % cand == 0:
            return cand
    return n


def _body(ptr_ref, ptrw_ref, x_ref, h_ref, w_ref, b_ref, out_ref, *, K, F, D, B):
    k = pl.program_id(0)

    @pl.when(k == 0)
    def _init():
        out_ref[...] = jnp.zeros_like(out_ref)

    r0 = k * K
    z = jnp.dot(x_ref[...], w_ref[:F], preferred_element_type=jnp.float32)
    z = z + jnp.dot(h_ref[...], w_ref[F:], preferred_element_type=jnp.float32)
    z = z + b_ref[...]
    z = jnp.where(z >= 0, z, 0.2 * z)

    ptr_wide = ptrw_ref[...]

    def seg_of(r):
        return jnp.sum((ptr_wide <= r).astype(jnp.int32)) - 1

    s0 = seg_of(r0)
    s1 = seg_of(r0 + K - 1)
    nchunks = (s1 - s0 + _SMAX) // _SMAX

    rows = r0 + jax.lax.broadcasted_iota(jnp.int32, (_SMAX, K), 1)

    def chunk(j, carry):
        base = s0 + j * _SMAX
        bounds = ptr_ref[pl.ds(base, _SMAX + 1), :]
        lo = bounds[:_SMAX]
        hi = bounds[1:]
        oh = ((rows >= lo) & (rows < hi)).astype(jnp.float32)
        part = jnp.dot(oh, z, preferred_element_type=jnp.float32)
        out_ref[pl.ds(base, _SMAX), :] += part
        return carry

    jax.lax.fori_loop(0, nchunks, chunk, 0)


def kernel(h_node, x, ptr, W, b):
    N, F = x.shape
    D = h_node.shape[1]
    B = ptr.shape[0] - 1
    K = _pick_block(N)
    G = N // K

    b_pad = -(B + _SMAX) % 8
    B_pad = B + _SMAX + b_pad
    P = -(B + 1 + _SMAX) % 8 + (B + 1 + _SMAX)
    ptr_pad = jnp.concatenate(
        [ptr.astype(jnp.int32), jnp.full((P - (B + 1),), N, jnp.int32)]
    ).reshape(P, 1)
    Pw = -(B + 1) % 1024 + (B + 1)
    ptr_wide = jnp.concatenate(
        [ptr.astype(jnp.int32), jnp.full((Pw - (B + 1),), N, jnp.int32)]
    ).reshape(Pw // 128, 128)

    out = pl.pallas_call(
        lambda *refs: _body(*refs, K=K, F=F, D=D, B=B),
        grid=(G,),
        in_specs=[
            pl.BlockSpec((P, 1), lambda k: (0, 0)),
            pl.BlockSpec((Pw // 128, 128), lambda k: (0, 0)),
            pl.BlockSpec((K, F), lambda k: (k, 0)),
            pl.BlockSpec((K, D), lambda k: (k, 0)),
            pl.BlockSpec((F + D, D), lambda k: (0, 0)),
            pl.BlockSpec((1, D), lambda k: (0, 0)),
        ],
        out_specs=pl.BlockSpec((B_pad, D), lambda k: (0, 0)),
        out_shape=jax.ShapeDtypeStruct((B_pad, D), jnp.float32),
        compiler_params=pltpu.CompilerParams(
            dimension_semantics=("arbitrary",),
        ),
    )(ptr_pad, ptr_wide, x, h_node, W, b.reshape(1, D))
    return out[:B]


# speedup vs baseline: 253.7971x; 1.6003x over previous
import jax
import jax.numpy as jnp
from jax.experimental import pallas as pl
from jax.experimental.pallas import tpu as pltpu

_SMAX = 32


def _pick_block(n):
    for cand in (2560, 1280, 640, 320, 160, 80, 40, 16, 8):
        if n % cand == 0:
            return cand
    return n


def _body(ptr_ref, ptrw_ref, x_ref, h_ref, w_ref, b_ref, out_ref, *, K, F, D, B):
    k = pl.program_id(0)

    @pl.when(k == 0)
    def _init():
        out_ref[...] = jnp.zeros_like(out_ref)

    r0 = k * K
    z = jnp.dot(x_ref[...], w_ref[:F], preferred_element_type=jnp.float32)
    z = z + jax.lax.dot_general(
        h_ref[...], w_ref[F:],
        dimension_numbers=(((0,), (0,)), ((), ())),
        preferred_element_type=jnp.float32,
    )
    z = z + b_ref[...]
    z = jnp.where(z >= 0, z, 0.2 * z)

    ptr_wide = ptrw_ref[...]

    def seg_of(r):
        return jnp.sum((ptr_wide <= r).astype(jnp.int32)) - 1

    s0 = seg_of(r0)
    s1 = seg_of(r0 + K - 1)
    nchunks = (s1 - s0 + _SMAX) // _SMAX

    rows = r0 + jax.lax.broadcasted_iota(jnp.int32, (_SMAX, K), 1)

    def chunk(j, carry):
        base = s0 + j * _SMAX
        bounds = ptr_ref[pl.ds(base, _SMAX + 1), :]
        lo = bounds[:_SMAX]
        hi = bounds[1:]
        oh = ((rows >= lo) & (rows < hi)).astype(jnp.float32)
        part = jnp.dot(oh, z, preferred_element_type=jnp.float32)
        out_ref[pl.ds(base, _SMAX), :] += part
        return carry

    jax.lax.fori_loop(0, nchunks, chunk, 0)


def kernel(h_node, x, ptr, W, b):
    N, F = x.shape
    D = h_node.shape[1]
    B = ptr.shape[0] - 1
    K = _pick_block(N)
    G = N // K

    b_pad = -(B + _SMAX) % 8
    B_pad = B + _SMAX + b_pad
    P = -(B + 1 + _SMAX) % 8 + (B + 1 + _SMAX)
    ptr_pad = jnp.concatenate(
        [ptr.astype(jnp.int32), jnp.full((P - (B + 1),), N, jnp.int32)]
    ).reshape(P, 1)
    Pw = -(B + 1) % 1024 + (B + 1)
    ptr_wide = jnp.concatenate(
        [ptr.astype(jnp.int32), jnp.full((Pw - (B + 1),), N, jnp.int32)]
    ).reshape(Pw // 128, 128)

    out = pl.pallas_call(
        lambda *refs: _body(*refs, K=K, F=F, D=D, B=B),
        grid=(G,),
        in_specs=[
            pl.BlockSpec((P, 1), lambda k: (0, 0)),
            pl.BlockSpec((Pw // 128, 128), lambda k: (0, 0)),
            pl.BlockSpec((K, F), lambda k: (k, 0)),
            pl.BlockSpec((D, K), lambda k: (0, k)),
            pl.BlockSpec((F + D, D), lambda k: (0, 0)),
            pl.BlockSpec((1, D), lambda k: (0, 0)),
        ],
        out_specs=pl.BlockSpec((B_pad, D), lambda k: (0, 0)),
        out_shape=jax.ShapeDtypeStruct((B_pad, D), jnp.float32),
        compiler_params=pltpu.CompilerParams(
            dimension_semantics=("arbitrary",),
        ),
    )(ptr_pad, ptr_wide, x, h_node.T, W, b.reshape(1, D))
    return out[:B]


# speedup vs baseline: 254.7302x; 1.0037x over previous
import jax
import jax.numpy as jnp
from jax.experimental import pallas as pl
from jax.experimental.pallas import tpu as pltpu

_SMAX = 32


def _pick_block(n):
    for cand in (2560, 1280, 640, 320, 160, 80, 40, 16, 8):
        if n % cand == 0:
            return cand
    return n


def _body(ptr_ref, ptrw_ref, x_ref, h_ref, w_ref, b_ref, out_ref, *, K, F, D, B):
    k = pl.program_id(0)

    @pl.when(k == 0)
    def _init():
        out_ref[...] = jnp.zeros_like(out_ref)

    r0 = k * K
    z = jnp.dot(x_ref[...], w_ref[:F], preferred_element_type=jnp.float32)
    z = z + jax.lax.dot_general(
        h_ref[...], w_ref[F:],
        dimension_numbers=(((0,), (0,)), ((), ())),
        preferred_element_type=jnp.float32,
    )
    z = z + b_ref[...]
    z = jnp.where(z >= 0, z, 0.2 * z)

    ptr_wide = ptrw_ref[...]

    def seg_of(r):
        return jnp.sum((ptr_wide <= r).astype(jnp.int32)) - 1

    s0 = seg_of(r0)
    s1 = seg_of(r0 + K - 1)
    nchunks = (s1 - s0 + _SMAX) // _SMAX

    rows = r0 + jax.lax.broadcasted_iota(jnp.int32, (_SMAX + 1, K), 1)

    def chunk(j, carry):
        base = s0 + j * _SMAX
        bounds = ptr_ref[pl.ds(base, _SMAX + 1), :]
        step = (rows >= bounds).astype(jnp.float32)
        csum = jnp.dot(step, z, preferred_element_type=jnp.float32)
        part = csum[:_SMAX] - csum[1:]
        out_ref[pl.ds(base, _SMAX), :] += part
        return carry

    jax.lax.fori_loop(0, nchunks, chunk, 0)


def kernel(h_node, x, ptr, W, b):
    N, F = x.shape
    D = h_node.shape[1]
    B = ptr.shape[0] - 1
    K = _pick_block(N)
    G = N // K

    b_pad = -(B + _SMAX) % 8
    B_pad = B + _SMAX + b_pad
    P = -(B + 1 + _SMAX) % 8 + (B + 1 + _SMAX)
    ptr_pad = jnp.concatenate(
        [ptr.astype(jnp.int32), jnp.full((P - (B + 1),), N, jnp.int32)]
    ).reshape(P, 1)
    Pw = -(B + 1) % 1024 + (B + 1)
    ptr_wide = jnp.concatenate(
        [ptr.astype(jnp.int32), jnp.full((Pw - (B + 1),), N, jnp.int32)]
    ).reshape(Pw // 128, 128)

    out = pl.pallas_call(
        lambda *refs: _body(*refs, K=K, F=F, D=D, B=B),
        grid=(G,),
        in_specs=[
            pl.BlockSpec((P, 1), lambda k: (0, 0)),
            pl.BlockSpec((Pw // 128, 128), lambda k: (0, 0)),
            pl.BlockSpec((K, F), lambda k: (k, 0)),
            pl.BlockSpec((D, K), lambda k: (0, k)),
            pl.BlockSpec((F + D, D), lambda k: (0, 0)),
            pl.BlockSpec((1, D), lambda k: (0, 0)),
        ],
        out_specs=pl.BlockSpec((B_pad, D), lambda k: (0, 0)),
        out_shape=jax.ShapeDtypeStruct((B_pad, D), jnp.float32),
        compiler_params=pltpu.CompilerParams(
            dimension_semantics=("arbitrary",),
        ),
    )(ptr_pad, ptr_wide, x, h_node.T, W, b.reshape(1, D))
    return out[:B]


# speedup vs baseline: 311.5541x; 1.2231x over previous
import jax
import jax.numpy as jnp
from jax.experimental import pallas as pl
from jax.experimental.pallas import tpu as pltpu

_SMAX = 32


def _pick_block(n):
    for cand in (5120, 2560, 1280, 640, 320, 160, 80, 40, 16, 8):
        if n % cand == 0:
            return cand
    return n


def _body(ptr_ref, ptrw_ref, x_ref, h_ref, w_ref, b_ref, out_ref, *, K, F, D, B):
    k = pl.program_id(0)

    @pl.when(k == 0)
    def _init():
        out_ref[...] = jnp.zeros_like(out_ref)

    r0 = k * K
    z = jnp.dot(
        x_ref[...].astype(jnp.bfloat16),
        w_ref[:F].astype(jnp.bfloat16),
        preferred_element_type=jnp.float32,
    )
    z = z + jax.lax.dot_general(
        h_ref[...], w_ref[F:],
        dimension_numbers=(((0,), (0,)), ((), ())),
        preferred_element_type=jnp.float32,
    )
    z = z + b_ref[...]
    z = jnp.where(z >= 0, z, 0.2 * z)

    ptr_wide = ptrw_ref[...]

    def seg_of(r):
        return jnp.sum((ptr_wide <= r).astype(jnp.int32)) - 1

    s0 = seg_of(r0)
    s1 = seg_of(r0 + K - 1)
    nchunks = (s1 - s0 + _SMAX) // _SMAX

    rows = r0 + jax.lax.broadcasted_iota(jnp.int32, (_SMAX + 1, K), 1)

    def chunk(j, carry):
        base = s0 + j * _SMAX
        bounds = ptr_ref[pl.ds(base, _SMAX + 1), :]
        step = (rows >= bounds).astype(jnp.float32)
        csum = jnp.dot(step, z, preferred_element_type=jnp.float32)
        part = csum[:_SMAX] - csum[1:]
        out_ref[pl.ds(base, _SMAX), :] += part
        return carry

    jax.lax.fori_loop(0, nchunks, chunk, 0)


def kernel(h_node, x, ptr, W, b):
    N, F = x.shape
    D = h_node.shape[1]
    B = ptr.shape[0] - 1
    K = _pick_block(N)
    G = N // K

    b_pad = -(B + _SMAX) % 8
    B_pad = B + _SMAX + b_pad
    P = -(B + 1 + _SMAX) % 8 + (B + 1 + _SMAX)
    ptr_pad = jnp.concatenate(
        [ptr.astype(jnp.int32), jnp.full((P - (B + 1),), N, jnp.int32)]
    ).reshape(P, 1)
    Pw = -(B + 1) % 1024 + (B + 1)
    ptr_wide = jnp.concatenate(
        [ptr.astype(jnp.int32), jnp.full((Pw - (B + 1),), N, jnp.int32)]
    ).reshape(Pw // 128, 128)

    out = pl.pallas_call(
        lambda *refs: _body(*refs, K=K, F=F, D=D, B=B),
        grid=(G,),
        in_specs=[
            pl.BlockSpec((P, 1), lambda k: (0, 0)),
            pl.BlockSpec((Pw // 128, 128), lambda k: (0, 0)),
            pl.BlockSpec((K, F), lambda k: (k, 0)),
            pl.BlockSpec((D, K), lambda k: (0, k)),
            pl.BlockSpec((F + D, D), lambda k: (0, 0)),
            pl.BlockSpec((1, D), lambda k: (0, 0)),
        ],
        out_specs=pl.BlockSpec((B_pad, D), lambda k: (0, 0)),
        out_shape=jax.ShapeDtypeStruct((B_pad, D), jnp.float32),
        compiler_params=pltpu.CompilerParams(
            dimension_semantics=("arbitrary",),
        ),
    )(ptr_pad, ptr_wide, x, h_node.T, W, b.reshape(1, D))
    return out[:B]
